# baseline (speedup 1.0000x reference)
import functools

import jax
import jax.numpy as jnp
from jax.experimental import pallas as pl
from jax.experimental.pallas import tpu as pltpu
from jax.experimental.pallas import tpu_sc as plsc

N = 8192
K = 8192
D = 64

NH = 4096
KB = 1024
NB = NH


def _argmin_body(wt_ref, xt_ref, out_ref, min_ref):
    k = pl.program_id(1)
    wtb = wt_ref[...]
    xtb = xt_ref[...]
    w2 = jnp.sum(wtb * wtb, axis=1, keepdims=True)
    x2 = jnp.sum(xtb * xtb, axis=0, keepdims=True)
    prod = jax.lax.dot_general(
        (wtb * -2.0).astype(jnp.bfloat16), xtb.astype(jnp.bfloat16),
        (((1,), (0,)), ((), ())),
        preferred_element_type=jnp.float32)
    d2 = jnp.maximum((x2 + prod) + w2, 0.0)
    tmin_d2 = jnp.min(d2, axis=0, keepdims=True)
    gidx = jax.lax.broadcasted_iota(jnp.int32, d2.shape, 0) + k * KB
    tidx = jnp.min(jnp.where(d2 == tmin_d2, gidx, jnp.int32(2**31 - 1)),
                   axis=0, keepdims=True)
    tmin = jnp.sqrt(tmin_d2)

    @pl.when(k == 0)
    def _():
        min_ref[...] = tmin
        out_ref[...] = tidx

    @pl.when(k != 0)
    def _():
        cur = min_ref[...]
        better = tmin < cur
        min_ref[...] = jnp.where(better, tmin, cur)
        out_ref[...] = jnp.where(better, tidx, out_ref[...])

    @pl.when(k == (K // 2) // KB - 1)
    def _():
        b = jax.lax.bitcast_convert_type(min_ref[...], jnp.uint32)
        b = (b + jnp.uint32(0x7FFF) + ((b >> 16) & jnp.uint32(1))) \
            & jnp.uint32(0xFFFF0000)
        min_ref[...] = jax.lax.bitcast_convert_type(b, jnp.float32)


def _tc_argmin(wT, xT):
    return pl.pallas_call(
        _argmin_body,
        grid=(NH // NB, K // KB),
        in_specs=[
            pl.BlockSpec((KB, D), lambda n, k: (k, 0)),
            pl.BlockSpec((D, NB), lambda n, k: (0, n)),
        ],
        out_specs=pl.BlockSpec((1, NB), lambda n, k: (0, n)),
        out_shape=jax.ShapeDtypeStruct((1, NH), jnp.int32),
        scratch_shapes=[pltpu.VMEM((1, NB), jnp.float32)],
        compiler_params=pltpu.CompilerParams(
            dimension_semantics=("parallel", "arbitrary")),
    )(wT, xT)


def _sc_gather(table, idx):
    mesh = plsc.VectorSubcoreMesh(core_axis_name="c", subcore_axis_name="s")
    n_workers = 32
    bpw = NH // n_workers
    half = bpw

    @functools.partial(
        pl.kernel,
        mesh=mesh,
        out_type=jax.ShapeDtypeStruct((NH, D), jnp.float32),
        scratch_types=[
            pltpu.VMEM((half,), jnp.int32),
            pltpu.VMEM((half, D), jnp.float32),
            pltpu.SemaphoreType.DMA,
        ],
        compiler_params=pltpu.CompilerParams(use_tc_tiling_on_sc=False),
    )
    def gather_kernel(table_hbm, idx_hbm, out_hbm, idx0, rows0, sem):
        wid = jax.lax.axis_index("s") * 2 + jax.lax.axis_index("c")
        base = wid * bpw
        pltpu.sync_copy(idx_hbm.at[pl.ds(base, half)], idx0)
        pltpu.async_copy(table_hbm.at[idx0], rows0, sem).wait()
        pltpu.sync_copy(rows0, out_hbm.at[pl.ds(base, half)])

    return gather_kernel(table, idx)


def kernel(x, weight):
    B, _, H, W = x.shape
    xT = x.transpose(1, 0, 2, 3).reshape(D, N)
    wT = weight.T
    idx0 = _tc_argmin(wT, xT[:, :NH]).reshape(NH)
    quant0 = _sc_gather(wT, idx0)
    idx1 = _tc_argmin(wT, xT[:, NH:]).reshape(NH)
    quant1 = _sc_gather(wT, idx1)
    idx = jnp.concatenate([idx0, idx1])
    quant = jnp.concatenate([quant0, quant1], axis=0)
    quant = quant.reshape(B, H, W, D).transpose(0, 3, 1, 2)
    result = x + (quant - x)
    return result, idx.reshape(B, H, W)


# speedup vs baseline: 1.1604x; 1.1604x over previous
import functools

import jax
import jax.numpy as jnp
from jax.experimental import pallas as pl
from jax.experimental.pallas import tpu as pltpu
from jax.experimental.pallas import tpu_sc as plsc

N = 8192
K = 8192
D = 64

NB = 8192
KB = 1024


def _argmin_body(wt_ref, xt_ref, out_ref, min_ref):
    k = pl.program_id(1)
    wtb = wt_ref[...]
    xtb = xt_ref[...]
    w2 = jnp.sum(wtb * wtb, axis=1, keepdims=True)
    x2 = jnp.sum(xtb * xtb, axis=0, keepdims=True)
    prod = jax.lax.dot_general(
        (wtb * -2.0).astype(jnp.bfloat16), xtb.astype(jnp.bfloat16),
        (((1,), (0,)), ((), ())),
        preferred_element_type=jnp.float32)
    d2 = (x2 + prod) + w2
    tmin_d2 = jnp.min(d2, axis=0, keepdims=True)
    gidx = jax.lax.broadcasted_iota(jnp.int32, d2.shape, 0) + k * KB
    tidx = jnp.min(jnp.where(d2 == tmin_d2, gidx, jnp.int32(2**31 - 1)),
                   axis=0, keepdims=True)
    tmin = jnp.sqrt(jnp.maximum(tmin_d2, 0.0))

    @pl.when(k == 0)
    def _():
        min_ref[...] = tmin
        out_ref[...] = tidx

    @pl.when(k != 0)
    def _():
        cur = min_ref[...]
        better = tmin < cur
        min_ref[...] = jnp.where(better, tmin, cur)
        out_ref[...] = jnp.where(better, tidx, out_ref[...])

    @pl.when(k == (K // 2) // KB - 1)
    def _():
        b = jax.lax.bitcast_convert_type(min_ref[...], jnp.uint32)
        b = (b + jnp.uint32(0x7FFF) + ((b >> 16) & jnp.uint32(1))) \
            & jnp.uint32(0xFFFF0000)
        min_ref[...] = jax.lax.bitcast_convert_type(b, jnp.float32)


def _tc_argmin(wT, xT):
    return pl.pallas_call(
        _argmin_body,
        grid=(N // NB, K // KB),
        in_specs=[
            pl.BlockSpec((KB, D), lambda n, k: (k, 0)),
            pl.BlockSpec((D, NB), lambda n, k: (0, n)),
        ],
        out_specs=pl.BlockSpec((1, NB), lambda n, k: (0, n)),
        out_shape=jax.ShapeDtypeStruct((1, N), jnp.int32),
        scratch_shapes=[pltpu.VMEM((1, NB), jnp.float32)],
        compiler_params=pltpu.CompilerParams(
            dimension_semantics=("parallel", "arbitrary")),
    )(wT, xT)


def _sc_gather(table, idx):
    mesh = plsc.VectorSubcoreMesh(core_axis_name="c", subcore_axis_name="s")
    n_workers = 32
    bpw = N // n_workers
    half = bpw // 2

    @functools.partial(
        pl.kernel,
        mesh=mesh,
        out_type=jax.ShapeDtypeStruct((N, D), jnp.float32),
        scratch_types=[
            pltpu.VMEM((half,), jnp.int32),
            pltpu.VMEM((half,), jnp.int32),
            pltpu.VMEM((half, D), jnp.float32),
            pltpu.VMEM((half, D), jnp.float32),
            pltpu.SemaphoreType.DMA,
        ],
        compiler_params=pltpu.CompilerParams(use_tc_tiling_on_sc=False),
    )
    def gather_kernel(table_hbm, idx_hbm, out_hbm, idx0, idx1, rows0, rows1,
                      sem):
        wid = jax.lax.axis_index("s") * 2 + jax.lax.axis_index("c")
        base = wid * bpw
        pltpu.sync_copy(idx_hbm.at[pl.ds(base, half)], idx0)
        pltpu.sync_copy(idx_hbm.at[pl.ds(base + half, half)], idx1)
        cp0 = pltpu.async_copy(table_hbm.at[idx0], rows0, sem)
        cp1 = pltpu.async_copy(table_hbm.at[idx1], rows1, sem)
        cp0.wait()
        cp1.wait()
        pltpu.sync_copy(rows0, out_hbm.at[pl.ds(base, half)])
        pltpu.sync_copy(rows1, out_hbm.at[pl.ds(base + half, half)])

    return gather_kernel(table, idx)


def kernel(x, weight):
    B, _, H, W = x.shape
    xT = x.transpose(1, 0, 2, 3).reshape(D, N)
    wT = weight.T
    idx = _tc_argmin(wT, xT).reshape(N)
    quant = _sc_gather(wT, idx)
    quant = quant.reshape(B, H, W, D).transpose(0, 3, 1, 2)
    result = x + (quant - x)
    return result, idx.reshape(B, H, W)
